# baseline (speedup 1.0000x reference)
import functools

import jax
import jax.numpy as jnp
from jax import lax
from jax.experimental import pallas as pl
from jax.experimental.pallas import tpu as pltpu
from jax.experimental.pallas import tpu_sc as plsc

B = 16384
D = 16
FEAT = 1000000
NC = 2
NS = 16
L = 16
NW = NC * NS
BPW = B // NW
IPW = 2 * BPW
ICHUNK = 128
NCHUNK = IPW // ICHUNK
EPC = ICHUNK // 2
GPC = EPC // L
WPC = D * ICHUNK


def _lfm_body(x_hbm, tw_hbm, out_hbm, idx_v, hi_v, buf_a, buf_b, out_v,
              sem_a, sem_b):
    wid = lax.axis_index("s") * NC + lax.axis_index("c")

    pltpu.sync_copy(x_hbm.at[pl.ds(wid * NCHUNK, NCHUNK)], idx_v)
    for t in range(NCHUNK):
        for d in range(D):
            for c in range(ICHUNK // L):
                hi_v[pl.ds(t * WPC + d * ICHUNK + c * L, L)] = (
                    idx_v[t, pl.ds(c * L, L)] + (d * FEAT)
                )

    bufs = [buf_a, buf_b]
    sems = [sem_a, sem_b]
    lanes = lax.iota(jnp.int32, L)

    def fire(j):
        return pltpu.async_copy(
            tw_hbm.at[hi_v.at[pl.ds(j * WPC, WPC)]], bufs[j % 2], sems[j % 2]
        )

    ks_tab = [((lanes + s) & (L - 1)) * ICHUNK for s in range(D)]

    def compute(j):
        buf = bufs[j % 2]
        for g in range(GPC):
            k0 = 2 * (g * L) + 2 * lanes
            k1 = k0 + 1
            acc = jnp.zeros((L,), jnp.float32)
            for s in range(D):
                a = plsc.load_gather(buf, [ks_tab[s] + k0])
                b = plsc.load_gather(buf, [ks_tab[s] + k1])
                acc = acc + a * b
            out_v[pl.ds(j * EPC + g * L, L)] = 1.0 / (1.0 + jnp.exp(-acc))

    copies = [fire(0), fire(1)]
    for j in range(NCHUNK):
        copies[j].wait()
        compute(j)
        if j + 2 < NCHUNK:
            copies.append(fire(j + 2))

    pltpu.sync_copy(out_v, out_hbm.at[pl.ds(wid * BPW, BPW)])


@functools.partial(
    pl.kernel,
    out_type=jax.ShapeDtypeStruct((B,), jnp.float32),
    mesh=plsc.VectorSubcoreMesh(core_axis_name="c", subcore_axis_name="s"),
    compiler_params=pltpu.CompilerParams(needs_layout_passes=False),
    scratch_types=[
        pltpu.VMEM((NCHUNK, ICHUNK), jnp.int32),
        pltpu.VMEM((NCHUNK * WPC,), jnp.int32),
        pltpu.VMEM((WPC,), jnp.float32),
        pltpu.VMEM((WPC,), jnp.float32),
        pltpu.VMEM((BPW,), jnp.float32),
        pltpu.SemaphoreType.DMA,
        pltpu.SemaphoreType.DMA,
    ],
)
def _lfm_sc(x_hbm, tw_hbm, out_hbm, idx_v, hi_v, buf_a, buf_b, out_v,
            sem_a, sem_b):
    _lfm_body(x_hbm, tw_hbm, out_hbm, idx_v, hi_v, buf_a, buf_b, out_v,
              sem_a, sem_b)


def kernel(x, table):
    x2 = x.astype(jnp.int32).reshape(NW * NCHUNK, ICHUNK)
    tw = table.T.reshape(D * FEAT)
    out = _lfm_sc(x2, tw)
    return out.reshape(B, 1)


# speedup vs baseline: 8.0868x; 8.0868x over previous
import functools

import jax
import jax.numpy as jnp
from jax import lax
from jax.experimental import pallas as pl
from jax.experimental.pallas import tpu as pltpu
from jax.experimental.pallas import tpu_sc as plsc

B = 16384
D = 16
FEAT = 1000000
PACK = 8
ROW128 = FEAT // PACK
NC = 2
NS = 16
L = 16
NW = NC * NS
BPW = B // NW
IPW = 2 * BPW
ICHUNK = 128
NCHUNK = IPW // ICHUNK
EPC = ICHUNK // 2
GPC = EPC // L

NCOL = FEAT // ICHUNK
CPW = NCOL // NW
MAINL = NCOL * L
BAT = 4
BW = BAT * ICHUNK
TBR = BAT * L
NBAT = NCOL // BAT
CPB = NBAT // NW
LEFTB = NW * CPB


def _retile_body(tt_hbm, tail_hbm, w2_hbm, buf_a, buf_b, tb_a, tb_b,
                 sin_a, sin_b, sout_a, sout_b):
    wid = lax.axis_index("s") * NC + lax.axis_index("c")
    base = wid * CPB
    lanes = lax.iota(jnp.int32, L)

    bufs = [buf_a, buf_b]
    tbs = [tb_a, tb_b]
    sins = [sin_a, sin_b]
    souts = [sout_a, sout_b]

    def start_in(b, p):
        pltpu.make_async_copy(
            tt_hbm.at[:, pl.ds(b * BW, BW)], bufs[p], sins[p]
        ).start()

    def wait_in(p):
        pltpu.make_async_copy(
            tt_hbm.at[:, pl.ds(0, BW)], bufs[p], sins[p]
        ).wait()

    def start_out(b, p):
        pltpu.make_async_copy(
            tbs[p], w2_hbm.at[pl.ds(b * TBR, TBR), :], souts[p]
        ).start()

    def wait_out(p):
        pltpu.make_async_copy(
            tbs[p], w2_hbm.at[pl.ds(0, TBR), :], souts[p]
        ).wait()

    def transpose_bat(p):
        buf, tb = bufs[p], tbs[p]
        colouts = [((lanes & (PACK - 1)) << 4) + ((d + lanes) & (L - 1))
                   for d in range(D)]
        rsh = lax.shift_right_logical(lanes, 3)

        def ccbody(cc, _):
            for k in range(ICHUNK // L):
                row = L * cc + 2 * k + rsh
                for d in range(D):
                    v = buf[d, pl.ds(cc * ICHUNK + L * k, L)]
                    plsc.store_scatter(tb, [row, colouts[d]], v)
            return 0

        lax.fori_loop(0, BAT, ccbody, 0)

    start_in(base, 0)

    def body(j, _):
        b0 = base + 2 * j
        start_in(b0 + 1, 1)
        wait_in(0)
        transpose_bat(0)

        @pl.when(j > 0)
        def _():
            wait_out(0)

        start_out(b0, 0)

        @pl.when(j < CPB // 2 - 1)
        def _():
            start_in(b0 + 2, 0)

        wait_in(1)
        transpose_bat(1)

        @pl.when(j > 0)
        def _():
            wait_out(1)

        start_out(b0 + 1, 1)
        return 0

    lax.fori_loop(0, CPB // 2, body, 0)
    wait_out(0)
    wait_out(1)

    def sync_batch(b):
        pltpu.sync_copy(tt_hbm.at[:, pl.ds(b * BW, BW)], buf_a)
        transpose_bat(0)
        pltpu.sync_copy(tb_a, w2_hbm.at[pl.ds(b * TBR, TBR), :])

    sync_batch(base + CPB - 1)

    @pl.when(wid == 0)
    def _():
        sync_batch(LEFTB)

    @pl.when(wid == NW - 1)
    def _():
        pltpu.sync_copy(tail_hbm, tb_a.at[pl.ds(0, PACK)])
        pltpu.sync_copy(tb_a.at[pl.ds(0, PACK)], w2_hbm.at[pl.ds(MAINL, PACK), :])


@functools.partial(
    pl.kernel,
    out_type=jax.ShapeDtypeStruct((ROW128, PACK * D), jnp.float32),
    mesh=plsc.VectorSubcoreMesh(core_axis_name="c", subcore_axis_name="s"),
    compiler_params=pltpu.CompilerParams(needs_layout_passes=False),
    scratch_types=[
        pltpu.VMEM((D, BW), jnp.float32),
        pltpu.VMEM((D, BW), jnp.float32),
        pltpu.VMEM((TBR, PACK * D), jnp.float32),
        pltpu.VMEM((TBR, PACK * D), jnp.float32),
        pltpu.SemaphoreType.DMA,
        pltpu.SemaphoreType.DMA,
        pltpu.SemaphoreType.DMA,
        pltpu.SemaphoreType.DMA,
    ],
)
def _retile_sc(tt_hbm, tail_hbm, w2_hbm, buf_a, buf_b, tb_a, tb_b,
               sin_a, sin_b, sout_a, sout_b):
    _retile_body(tt_hbm, tail_hbm, w2_hbm, buf_a, buf_b, tb_a, tb_b,
                 sin_a, sin_b, sout_a, sout_b)


def _lfm_body(x_hbm, table_hbm, out_hbm, idx_v, hi_v, buf_a, buf_b, out_v,
              sem_a, sem_b):
    wid = lax.axis_index("s") * NC + lax.axis_index("c")

    pltpu.sync_copy(x_hbm.at[pl.ds(wid * NCHUNK, NCHUNK)], idx_v)
    for t in range(NCHUNK):
        for c in range(ICHUNK // L):
            hi_v[t, pl.ds(c * L, L)] = lax.shift_right_logical(
                idx_v[t, pl.ds(c * L, L)], 3
            )

    bufs = [buf_a, buf_b]
    sems = [sem_a, sem_b]
    lanes = lax.iota(jnp.int32, L)

    def fire(j):
        return pltpu.async_copy(
            table_hbm.at[hi_v.at[j]], bufs[j % 2], sems[j % 2]
        )

    ks_tab = [(lanes + s) & (L - 1) for s in range(D)]

    def compute(j):
        buf = bufs[j % 2]
        jvec = jnp.full((L,), j, jnp.int32)
        for g in range(GPC):
            k0 = 2 * (g * L) + 2 * lanes
            k1 = k0 + 1
            i0 = plsc.load_gather(idx_v, [jvec, k0])
            i1 = plsc.load_gather(idx_v, [jvec, k1])
            c0 = (i0 & 7) * D
            c1 = (i1 & 7) * D
            m = (i1 - i0) & (L - 1)
            acc = jnp.zeros((L,), jnp.float32)
            for s in range(D):
                a = plsc.load_gather(buf, [k0, c0 + ks_tab[s]])
                b = plsc.load_gather(buf, [k1, c1 + ((ks_tab[s] + m) & (L - 1))])
                acc = acc + a * b
            out_v[pl.ds(j * EPC + g * L, L)] = 1.0 / (1.0 + jnp.exp(-acc))

    copies = [fire(0), fire(1)]
    for j in range(NCHUNK):
        copies[j].wait()
        compute(j)
        if j + 2 < NCHUNK:
            copies.append(fire(j + 2))

    pltpu.sync_copy(out_v, out_hbm.at[pl.ds(wid * BPW, BPW)])


@functools.partial(
    pl.kernel,
    out_type=jax.ShapeDtypeStruct((B,), jnp.float32),
    mesh=plsc.VectorSubcoreMesh(core_axis_name="c", subcore_axis_name="s"),
    compiler_params=pltpu.CompilerParams(needs_layout_passes=False),
    scratch_types=[
        pltpu.VMEM((NCHUNK, ICHUNK), jnp.int32),
        pltpu.VMEM((NCHUNK, ICHUNK), jnp.int32),
        pltpu.VMEM((ICHUNK, PACK * D), jnp.float32),
        pltpu.VMEM((ICHUNK, PACK * D), jnp.float32),
        pltpu.VMEM((BPW,), jnp.float32),
        pltpu.SemaphoreType.DMA,
        pltpu.SemaphoreType.DMA,
    ],
)
def _lfm_sc(x_hbm, table_hbm, out_hbm, idx_v, hi_v, buf_a, buf_b, out_v,
            sem_a, sem_b):
    _lfm_body(x_hbm, table_hbm, out_hbm, idx_v, hi_v, buf_a, buf_b, out_v,
              sem_a, sem_b)


def kernel(x, table):
    x2 = x.astype(jnp.int32).reshape(NW * NCHUNK, ICHUNK)
    tt = table.T
    tailm = table[NCOL * ICHUNK:]
    j64 = jnp.arange(64, dtype=jnp.int32)[:, None]
    p16 = jnp.arange(D, dtype=jnp.int32)[None, :]
    tail8 = jnp.take_along_axis(tailm, (p16 - j64) % D, axis=1)
    tail8 = tail8.reshape(PACK, PACK * D)
    t128 = _retile_sc(tt, tail8)
    out = _lfm_sc(x2, t128)
    return out.reshape(B, 1)
